# baseline (speedup 1.0000x reference)
import functools

import jax
import jax.numpy as jnp
from jax import lax
from jax.experimental import pallas as pl
from jax.experimental.pallas import tpu as pltpu
from jax.experimental.pallas import tpu_sc as plsc

_NUM_TOKENS = 4096 * 200
_FEATURES = 64
_NW = 32
_PER_W = _NUM_TOKENS // _NW
_K = 512
_KSUB = 1
_C = _K * _KSUB
_NSLOT = 2
_NCHUNK = _PER_W // _C
_NJ = _NCHUNK // _NSLOT
_NROWS = _PER_W // _K


def _make_kernel():
    mesh = plsc.VectorSubcoreMesh(core_axis_name="c", subcore_axis_name="s")

    @functools.partial(
        pl.kernel,
        mesh=mesh,
        compiler_params=pltpu.CompilerParams(use_tc_tiling_on_sc=False),
        out_type=jax.ShapeDtypeStruct((_NUM_TOKENS, _FEATURES), jnp.float32),
        scratch_types=[
            pltpu.VMEM((_NROWS, _K), jnp.int32),
        ]
        + [pltpu.VMEM((_C, _FEATURES), jnp.float32) for _ in range(_NSLOT)]
        + [pltpu.SemaphoreType.DMA for _ in range(2 * _NSLOT)],
    )
    def emb_kernel(idx_hbm, table_hbm, out_hbm, idx_v, *rest):
        bufs = rest[:_NSLOT]
        gsems = rest[_NSLOT:2 * _NSLOT]
        wsems = rest[2 * _NSLOT:]

        wid = lax.axis_index("s") * 2 + lax.axis_index("c")
        base = wid * _PER_W

        pltpu.sync_copy(idx_hbm.at[wid], idx_v)

        def fire(c, buf, gsem):
            for s in range(_KSUB):
                pltpu.async_copy(
                    table_hbm.at[idx_v.at[c * _KSUB + s]],
                    buf.at[pl.ds(s * _K, _K)],
                    gsem,
                )

        def drain_gather(buf, gsem):
            for s in range(_KSUB):
                pltpu.make_async_copy(
                    table_hbm.at[idx_v.at[0]],
                    buf.at[pl.ds(s * _K, _K)],
                    gsem,
                ).wait()

        def start_write(c, buf, wsem):
            pltpu.async_copy(buf, out_hbm.at[pl.ds(base + c * _C, _C)], wsem)

        def drain_write(buf, wsem):
            pltpu.make_async_copy(buf, out_hbm.at[pl.ds(base, _C)], wsem).wait()

        def body(j, carry):
            for s in range(_NSLOT):
                c = _NSLOT * j + s

                @pl.when(j > 0)
                def _(s=s, wsem=wsems[s], buf=bufs[s]):
                    drain_write(buf, wsem)

                fire(c, bufs[s], gsems[s])
            for s in range(_NSLOT):
                c = _NSLOT * j + s
                drain_gather(bufs[s], gsems[s])
                start_write(c, bufs[s], wsems[s])
            return carry

        lax.fori_loop(0, _NJ, body, 0)
        for s in range(_NSLOT):
            drain_write(bufs[s], wsems[s])

    return emb_kernel


_emb = _make_kernel()


def kernel(tokens, embedding):
    idx = tokens.reshape(_NW, _NROWS, _K)
    out = _emb(idx, embedding)
    return out.reshape(tokens.shape[0], tokens.shape[1], _FEATURES)


# speedup vs baseline: 1.0052x; 1.0052x over previous
import functools

import jax
import jax.numpy as jnp
from jax import lax
from jax.experimental import pallas as pl
from jax.experimental.pallas import tpu as pltpu
from jax.experimental.pallas import tpu_sc as plsc

_NUM_TOKENS = 4096 * 200
_FEATURES = 64
_NW = 32
_PER_W = _NUM_TOKENS // _NW
_GPC = 32
_C = _GPC * 16
_NCHUNK = _PER_W // _C
_NSLOT = 2
_NJ = _NCHUNK // _NSLOT
_G = _PER_W // 16


def _make_kernel():
    mesh = plsc.VectorSubcoreMesh(core_axis_name="c", subcore_axis_name="s")

    @functools.partial(
        pl.kernel,
        mesh=mesh,
        compiler_params=pltpu.CompilerParams(use_tc_tiling_on_sc=False),
        out_type=jax.ShapeDtypeStruct((_NUM_TOKENS, _FEATURES), jnp.float32),
        scratch_types=[
            pltpu.VMEM((_G, 16), jnp.int32),
            pltpu.VMEM((_C, _FEATURES), jnp.float32),
            pltpu.VMEM((_C, _FEATURES), jnp.float32),
            pltpu.SemaphoreType.DMA,
            pltpu.SemaphoreType.DMA,
            pltpu.SemaphoreType.DMA,
            pltpu.SemaphoreType.DMA,
        ],
    )
    def emb_kernel(idx_hbm, table_hbm, out_hbm, idx_v, bufa, bufb, g0, g1, w0, w1):
        bufs = (bufa, bufb)
        gsems = (g0, g1)
        wsems = (w0, w1)
        wid = lax.axis_index("s") * 2 + lax.axis_index("c")
        base = wid * _PER_W

        pltpu.sync_copy(idx_hbm.at[wid], idx_v)

        def fire(c, buf, gsem):
            for g in range(_GPC):
                vals = idx_v[c * _GPC + g]
                pltpu.async_copy(
                    table_hbm.at[vals],
                    buf.at[pl.ds(g * 16, 16)],
                    gsem,
                )

        def drain_gather(c, buf, gsem):
            for g in range(_GPC):
                vals = idx_v[c * _GPC]
                pltpu.make_async_copy(
                    table_hbm.at[vals],
                    buf.at[pl.ds(g * 16, 16)],
                    gsem,
                ).wait()

        def body(j, carry):
            for s in range(_NSLOT):
                c = _NSLOT * j + s

                @pl.when(j > 0)
                def _(s=s, wsem=wsems[s], buf=bufs[s]):
                    pltpu.make_async_copy(
                        buf, out_hbm.at[pl.ds(base, _C)], wsem
                    ).wait()

                fire(c, bufs[s], gsems[s])
            for s in range(_NSLOT):
                c = _NSLOT * j + s
                drain_gather(c, bufs[s], gsems[s])
                pltpu.async_copy(
                    bufs[s], out_hbm.at[pl.ds(base + c * _C, _C)], wsems[s]
                )
            return carry

        lax.fori_loop(0, _NJ, body, 0)
        for s in range(_NSLOT):
            pltpu.make_async_copy(
                bufs[s], out_hbm.at[pl.ds(base, _C)], wsems[s]
            ).wait()

    return emb_kernel


_emb = _make_kernel()


def kernel(tokens, embedding):
    idx = tokens.reshape(_NW, _G, 16)
    out = _emb(idx, embedding)
    return out.reshape(tokens.shape[0], tokens.shape[1], _FEATURES)
